# baseline (speedup 1.0000x reference)
import functools

import jax
import jax.numpy as jnp
from jax import lax
from jax.experimental import pallas as pl
from jax.experimental.pallas import tpu as pltpu
from jax.experimental.pallas import tpu_sc as plsc

N = 10000
E = 320000
D = 128
NB = 5
NCMB = NB * NB * NB

NC = 2
NS = 16
NW = NC * NS
Q = E // NW
C = 80
IB = 25
NBLK = Q // (IB * C)
NPT = 624
NREM = N - NS * NPT


def _sc_agg_body(src_h, dst_h, pk_h, h_h, comb_h, out_h,
                 idx_s, idx_d, rows0, rows1, pk0, pk1, comb_v, agg_sh,
                 sg0, sg1, ss0, ss1, sp0, sp1):
    c = lax.axis_index("c")
    s = lax.axis_index("s")
    wid = c * NS + s

    pltpu.sync_copy(comb_h, comb_v)

    zero = jnp.zeros((16,), jnp.float32)

    def zrow(j, _):
        for k in range(D // 16):
            rows0[j, pl.ds(k * 16, 16)] = zero
        return 0

    lax.fori_loop(0, C, zrow, 0)

    def zcopy(j, _):
        pltpu.sync_copy(rows0, agg_sh.at[pl.ds(s * NPT + j * C, C)])
        return 0

    lax.fori_loop(0, NPT // C, zcopy, 0)
    pltpu.sync_copy(rows0.at[pl.ds(0, NPT - (NPT // C) * C)],
                    agg_sh.at[pl.ds(s * NPT + (NPT // C) * C,
                                    NPT - (NPT // C) * C)])

    @pl.when(s == 0)
    def _():
        pltpu.sync_copy(rows0.at[pl.ds(0, NREM)],
                        agg_sh.at[pl.ds(NS * NPT, NREM)])

    plsc.subcore_barrier()

    def compute_msgs(rows_v, pk_v):
        def group_body(g, _):
            for jj in range(16):
                j = g * 16 + jj
                wv = plsc.bitcast(pk_v[j, 0], jnp.float32)
                bv = pk_v[j, 1]
                for k in range(D // 16):
                    sl = pl.ds(k * 16, 16)
                    r = rows_v[j, sl]
                    cv = plsc.load_gather(comb_v, [bv + (k * 16)])
                    rows_v[j, sl] = jnp.maximum(r + cv, 0.0) * wv
            return 0

        lax.fori_loop(0, C // 16, group_body, 0)

    def gather(cg, i, rows_v, pk_v, semr, semp):
        pltpu.async_copy(h_h.at[idx_s.at[i]], rows_v, semr)
        pltpu.async_copy(pk_h.at[wid, cg], pk_v, semp)

    def scatter(i, rows_v, sem):
        return pltpu.async_copy(rows_v, agg_sh.at[idx_d.at[i]], sem,
                                add=True)

    def wait_gather(cg, i, rows_v, pk_v, semr, semp):
        pltpu.make_async_copy(h_h.at[idx_s.at[i]], rows_v, semr).wait()
        pltpu.make_async_copy(pk_h.at[wid, cg], pk_v, semp).wait()

    def wait_scatter(i, rows_v, sem):
        pltpu.make_async_copy(rows_v, agg_sh.at[idx_d.at[i]], sem).wait()

    def block_body(b, _):
        pltpu.sync_copy(src_h.at[wid, b], idx_s)
        pltpu.sync_copy(dst_h.at[wid, b], idx_d)
        cg0 = b * IB

        gather(cg0, 0, rows0, pk0, sg0, sp0)

        def pair_body(p, _):
            i0 = 2 * p
            wait_gather(cg0 + i0, i0, rows0, pk0, sg0, sp0)

            @pl.when(p > 0)
            def _():
                wait_scatter(i0 - 1, rows1, ss1)

            gather(cg0 + i0 + 1, i0 + 1, rows1, pk1, sg1, sp1)
            compute_msgs(rows0, pk0)
            scatter(i0, rows0, ss0)
            wait_gather(cg0 + i0 + 1, i0 + 1, rows1, pk1, sg1, sp1)

            @pl.when(i0 + 2 < IB)
            def _():
                wait_scatter(i0, rows0, ss0)
                gather(cg0 + i0 + 2, i0 + 2, rows0, pk0, sg0, sp0)

            compute_msgs(rows1, pk1)
            scatter(i0 + 1, rows1, ss1)
            return 0

        lax.fori_loop(0, (IB - 1) // 2, pair_body, 0)
        wait_gather(cg0 + IB - 1, IB - 1, rows0, pk0, sg0, sp0)
        compute_msgs(rows0, pk0)
        scatter(IB - 1, rows0, ss0)
        wait_scatter(IB - 2, rows1, ss1)
        wait_scatter(IB - 1, rows0, ss0)
        return 0

    lax.fori_loop(0, NBLK, block_body, 0)
    plsc.subcore_barrier()
    pltpu.sync_copy(agg_sh.at[pl.ds(s * NPT, NPT)],
                    out_h.at[c, pl.ds(s * NPT, NPT)])

    @pl.when(s == 0)
    def _():
        pltpu.sync_copy(agg_sh.at[pl.ds(NS * NPT, NREM)],
                        out_h.at[c, pl.ds(NS * NPT, NREM)])


_sc_agg = pl.kernel(
    _sc_agg_body,
    out_type=jax.ShapeDtypeStruct((NC, N, D), jnp.float32),
    mesh=plsc.VectorSubcoreMesh(core_axis_name="c", subcore_axis_name="s",
                                num_cores=NC, num_subcores=NS),
    scratch_types=[
        pltpu.VMEM((IB, C), jnp.int32),
        pltpu.VMEM((IB, C), jnp.int32),
        pltpu.VMEM((C, D), jnp.float32),
        pltpu.VMEM((C, D), jnp.float32),
        pltpu.VMEM((C, 2, 16), jnp.int32),
        pltpu.VMEM((C, 2, 16), jnp.int32),
        pltpu.VMEM((NCMB * D + 3 * D,), jnp.float32),
        pltpu.VMEM_SHARED((N, D), jnp.float32),
        pltpu.SemaphoreType.DMA,
        pltpu.SemaphoreType.DMA,
        pltpu.SemaphoreType.DMA,
        pltpu.SemaphoreType.DMA,
        pltpu.SemaphoreType.DMA,
        pltpu.SemaphoreType.DMA,
    ],
    compiler_params=pltpu.CompilerParams(use_tc_tiling_on_sc=False,
                                         needs_layout_passes=False),
)


def _comb_body(bt_ref, out_ref):
    nl = bt_ref.shape[0]
    for l in range(nl):
        b0 = bt_ref[l, 0]
        b1 = bt_ref[l, 1]
        b2 = bt_ref[l, 2]
        t = (b0[:, None, None, :] + b1[None, :, None, :]
             + b2[None, None, :, :]).reshape(NCMB, D)
        out_ref[l] = jnp.concatenate([t, jnp.zeros((3, D), jnp.float32)],
                                     axis=0)


def _build_comb(bond_tab):
    nl = bond_tab.shape[0]
    return pl.pallas_call(
        _comb_body,
        out_shape=jax.ShapeDtypeStruct((nl, NCMB + 3, D), jnp.float32),
    )(bond_tab)


def _mlp_body(last, h_ref, agg_ref, w1_ref, b1_ref, g1_ref, bb1_ref,
              w2_ref, b2_ref, go_ref, bo_ref, eps_ref, out_ref):
    h = h_ref[...]
    z = (1.0 + eps_ref[0]) * h + agg_ref[0] + agg_ref[1]
    u = jnp.dot(z, w1_ref[...], preferred_element_type=jnp.float32) + b1_ref[...]
    mu = jnp.mean(u, axis=0, keepdims=True)
    var = jnp.mean((u - mu) * (u - mu), axis=0, keepdims=True)
    u = (u - mu) * lax.rsqrt(var + 1e-5) * g1_ref[...] + bb1_ref[...]
    u = jnp.maximum(u, 0.0)
    v = jnp.dot(u, w2_ref[...], preferred_element_type=jnp.float32) + b2_ref[...]
    mu2 = jnp.mean(v, axis=0, keepdims=True)
    var2 = jnp.mean((v - mu2) * (v - mu2), axis=0, keepdims=True)
    v = (v - mu2) * lax.rsqrt(var2 + 1e-5) * go_ref[...] + bo_ref[...]
    if not last:
        v = jnp.maximum(v, 0.0)
    out_ref[...] = v


def _mlp(h, agg, w1, b1, g1, bb1, w2, b2, go, bo, eps_l, last):
    n, d = h.shape
    return pl.pallas_call(
        functools.partial(_mlp_body, last),
        out_shape=jax.ShapeDtypeStruct((n, d), jnp.float32),
        in_specs=[pl.BlockSpec(memory_space=pltpu.VMEM)] * 10
                 + [pl.BlockSpec(memory_space=pltpu.SMEM)],
    )(h, agg, w1, b1, g1, bb1, w2, b2, go, bo, eps_l)


def kernel(x, edge_index, edge_attr, edge_weights, bond_tab, W1, b1,
           bn1_g, bn1_b, W2, b2, eps, obn_g, obn_b):
    nl = W1.shape[0]
    src = edge_index[0]
    dst = edge_index[1]
    cmb = (edge_attr[:, 0] * NB + edge_attr[:, 1]) * NB + edge_attr[:, 2]

    src3 = src.reshape(NW, NBLK, IB, C)
    dst3 = dst.reshape(NW, NBLK, IB, C)
    iota16 = jnp.arange(16, dtype=jnp.int32)
    wbits = jax.lax.bitcast_convert_type(edge_weights, jnp.int32)
    bexp = cmb[:, None] * D + iota16[None, :]
    pk = jnp.stack(
        [jnp.broadcast_to(wbits[:, None], (E, 16)), bexp], axis=1)
    pk = pk.reshape(NW, NBLK * IB, C, 2, 16)

    comb = _build_comb(bond_tab).reshape(nl, (NCMB + 3) * D)

    h = x
    for l in range(nl):
        agg = _sc_agg(src3, dst3, pk, h, comb[l])
        h = _mlp(h, agg, W1[l], b1[l][None, :], bn1_g[l][None, :],
                 bn1_b[l][None, :], W2[l], b2[l][None, :], obn_g[l][None, :],
                 obn_b[l][None, :], eps[l:l + 1], last=(l == nl - 1))
    return h


# speedup vs baseline: 1.9061x; 1.9061x over previous
import functools

import jax
import jax.numpy as jnp
from jax import lax
from jax.experimental import pallas as pl
from jax.experimental.pallas import tpu as pltpu
from jax.experimental.pallas import tpu_sc as plsc

N = 10000
E = 320000
D = 128
NB = 5
NCMB = NB * NB * NB

NC = 2
NS = 16
NW = NC * NS
Q = E // NW
C = 80
IB = 25
NBLK = Q // (IB * C)
NPT = 624
NREM = N - NS * NPT


def _sc_agg_body(src_h, dst_h, pk_h, h_h, comb_h, out_h,
                 idx_s, idx_d, rows0, rows1, pk0, pk1, comb_v, agg_sh,
                 sg0, sg1, ss0, ss1, sp0, sp1):
    c = lax.axis_index("c")
    s = lax.axis_index("s")
    wid = c * NS + s

    pltpu.sync_copy(comb_h, comb_v)

    zero = jnp.zeros((16,), jnp.float32)

    def zrow(j, _):
        for k in range(D // 16):
            rows0[j, pl.ds(k * 16, 16)] = zero
        return 0

    lax.fori_loop(0, C, zrow, 0)

    def zcopy(j, _):
        pltpu.sync_copy(rows0, agg_sh.at[pl.ds(s * NPT + j * C, C)])
        return 0

    lax.fori_loop(0, NPT // C, zcopy, 0)
    pltpu.sync_copy(rows0.at[pl.ds(0, NPT - (NPT // C) * C)],
                    agg_sh.at[pl.ds(s * NPT + (NPT // C) * C,
                                    NPT - (NPT // C) * C)])

    @pl.when(s == 0)
    def _():
        pltpu.sync_copy(rows0.at[pl.ds(0, NREM)],
                        agg_sh.at[pl.ds(NS * NPT, NREM)])

    plsc.subcore_barrier()

    def compute_msgs(rows_v, pk_v):
        @plsc.parallel_loop(0, C, 1, unroll=8)
        def _(j):
            wv = plsc.bitcast(pk_v[pl.ds(j * 32, 16)], jnp.float32)
            bv = pk_v[pl.ds(j * 32 + 16, 16)]
            for k in range(D // 16):
                sl = pl.ds(k * 16, 16)
                r = rows_v[j, sl]
                cv = plsc.load_gather(comb_v, [bv + (k * 16)])
                rows_v[j, sl] = jnp.maximum(r + cv, 0.0) * wv

    def gather(cg, i, rows_v, pk_v, semr, semp):
        pltpu.async_copy(h_h.at[idx_s.at[i]], rows_v, semr)
        pltpu.async_copy(pk_h.at[wid, cg], pk_v, semp)

    def scatter(i, rows_v, sem):
        return pltpu.async_copy(rows_v, agg_sh.at[idx_d.at[i]], sem,
                                add=True)

    def wait_gather(cg, i, rows_v, pk_v, semr, semp):
        pltpu.make_async_copy(h_h.at[idx_s.at[i]], rows_v, semr).wait()
        pltpu.make_async_copy(pk_h.at[wid, cg], pk_v, semp).wait()

    def wait_scatter(i, rows_v, sem):
        pltpu.make_async_copy(rows_v, agg_sh.at[idx_d.at[i]], sem).wait()

    def block_body(b, _):
        pltpu.sync_copy(src_h.at[wid, b], idx_s)
        pltpu.sync_copy(dst_h.at[wid, b], idx_d)
        cg0 = b * IB

        gather(cg0, 0, rows0, pk0, sg0, sp0)

        def pair_body(p, _):
            i0 = 2 * p
            wait_gather(cg0 + i0, i0, rows0, pk0, sg0, sp0)

            @pl.when(p > 0)
            def _():
                wait_scatter(i0 - 1, rows1, ss1)

            gather(cg0 + i0 + 1, i0 + 1, rows1, pk1, sg1, sp1)
            compute_msgs(rows0, pk0)
            scatter(i0, rows0, ss0)
            wait_gather(cg0 + i0 + 1, i0 + 1, rows1, pk1, sg1, sp1)

            @pl.when(i0 + 2 < IB)
            def _():
                wait_scatter(i0, rows0, ss0)
                gather(cg0 + i0 + 2, i0 + 2, rows0, pk0, sg0, sp0)

            compute_msgs(rows1, pk1)
            scatter(i0 + 1, rows1, ss1)
            return 0

        lax.fori_loop(0, (IB - 1) // 2, pair_body, 0)
        wait_gather(cg0 + IB - 1, IB - 1, rows0, pk0, sg0, sp0)
        compute_msgs(rows0, pk0)
        scatter(IB - 1, rows0, ss0)
        wait_scatter(IB - 2, rows1, ss1)
        wait_scatter(IB - 1, rows0, ss0)
        return 0

    lax.fori_loop(0, NBLK, block_body, 0)
    plsc.subcore_barrier()
    pltpu.sync_copy(agg_sh.at[pl.ds(s * NPT, NPT)],
                    out_h.at[c, pl.ds(s * NPT, NPT)])

    @pl.when(s == 0)
    def _():
        pltpu.sync_copy(agg_sh.at[pl.ds(NS * NPT, NREM)],
                        out_h.at[c, pl.ds(NS * NPT, NREM)])


_sc_agg = pl.kernel(
    _sc_agg_body,
    out_type=jax.ShapeDtypeStruct((NC, N, D), jnp.float32),
    mesh=plsc.VectorSubcoreMesh(core_axis_name="c", subcore_axis_name="s",
                                num_cores=NC, num_subcores=NS),
    scratch_types=[
        pltpu.VMEM((IB, C), jnp.int32),
        pltpu.VMEM((IB, C), jnp.int32),
        pltpu.VMEM((C, D), jnp.float32),
        pltpu.VMEM((C, D), jnp.float32),
        pltpu.VMEM((C * 2 * 16,), jnp.int32),
        pltpu.VMEM((C * 2 * 16,), jnp.int32),
        pltpu.VMEM((NCMB * D + 3 * D,), jnp.float32),
        pltpu.VMEM_SHARED((N, D), jnp.float32),
        pltpu.SemaphoreType.DMA,
        pltpu.SemaphoreType.DMA,
        pltpu.SemaphoreType.DMA,
        pltpu.SemaphoreType.DMA,
        pltpu.SemaphoreType.DMA,
        pltpu.SemaphoreType.DMA,
    ],
    compiler_params=pltpu.CompilerParams(use_tc_tiling_on_sc=False,
                                         needs_layout_passes=False),
)


def _comb_body(bt_ref, out_ref):
    nl = bt_ref.shape[0]
    for l in range(nl):
        b0 = bt_ref[l, 0]
        b1 = bt_ref[l, 1]
        b2 = bt_ref[l, 2]
        t = (b0[:, None, None, :] + b1[None, :, None, :]
             + b2[None, None, :, :]).reshape(NCMB, D)
        out_ref[l] = jnp.concatenate([t, jnp.zeros((3, D), jnp.float32)],
                                     axis=0)


def _build_comb(bond_tab):
    nl = bond_tab.shape[0]
    return pl.pallas_call(
        _comb_body,
        out_shape=jax.ShapeDtypeStruct((nl, NCMB + 3, D), jnp.float32),
    )(bond_tab)


def _mlp_body(last, h_ref, agg_ref, w1_ref, b1_ref, g1_ref, bb1_ref,
              w2_ref, b2_ref, go_ref, bo_ref, eps_ref, out_ref):
    h = h_ref[...]
    z = (1.0 + eps_ref[0]) * h + agg_ref[0] + agg_ref[1]
    u = jnp.dot(z, w1_ref[...], preferred_element_type=jnp.float32) + b1_ref[...]
    mu = jnp.mean(u, axis=0, keepdims=True)
    var = jnp.mean((u - mu) * (u - mu), axis=0, keepdims=True)
    u = (u - mu) * lax.rsqrt(var + 1e-5) * g1_ref[...] + bb1_ref[...]
    u = jnp.maximum(u, 0.0)
    v = jnp.dot(u, w2_ref[...], preferred_element_type=jnp.float32) + b2_ref[...]
    mu2 = jnp.mean(v, axis=0, keepdims=True)
    var2 = jnp.mean((v - mu2) * (v - mu2), axis=0, keepdims=True)
    v = (v - mu2) * lax.rsqrt(var2 + 1e-5) * go_ref[...] + bo_ref[...]
    if not last:
        v = jnp.maximum(v, 0.0)
    out_ref[...] = v


def _mlp(h, agg, w1, b1, g1, bb1, w2, b2, go, bo, eps_l, last):
    n, d = h.shape
    return pl.pallas_call(
        functools.partial(_mlp_body, last),
        out_shape=jax.ShapeDtypeStruct((n, d), jnp.float32),
        in_specs=[pl.BlockSpec(memory_space=pltpu.VMEM)] * 10
                 + [pl.BlockSpec(memory_space=pltpu.SMEM)],
    )(h, agg, w1, b1, g1, bb1, w2, b2, go, bo, eps_l)


def kernel(x, edge_index, edge_attr, edge_weights, bond_tab, W1, b1,
           bn1_g, bn1_b, W2, b2, eps, obn_g, obn_b):
    nl = W1.shape[0]
    src = edge_index[0]
    dst = edge_index[1]
    cmb = (edge_attr[:, 0] * NB + edge_attr[:, 1]) * NB + edge_attr[:, 2]

    src3 = src.reshape(NW, NBLK, IB, C)
    dst3 = dst.reshape(NW, NBLK, IB, C)
    iota16 = jnp.arange(16, dtype=jnp.int32)
    wbits = jax.lax.bitcast_convert_type(edge_weights, jnp.int32)
    bexp = cmb[:, None] * D + iota16[None, :]
    pk = jnp.stack(
        [jnp.broadcast_to(wbits[:, None], (E, 16)), bexp], axis=1)
    pk = pk.reshape(NW, NBLK * IB, C * 2 * 16)

    comb = _build_comb(bond_tab).reshape(nl, (NCMB + 3) * D)

    h = x
    for l in range(nl):
        agg = _sc_agg(src3, dst3, pk, h, comb[l])
        h = _mlp(h, agg, W1[l], b1[l][None, :], bn1_g[l][None, :],
                 bn1_b[l][None, :], W2[l], b2[l][None, :], obn_g[l][None, :],
                 obn_b[l][None, :], eps[l:l + 1], last=(l == nl - 1))
    return h


# speedup vs baseline: 2.4187x; 1.2689x over previous
import functools

import jax
import jax.numpy as jnp
from jax import lax
from jax.experimental import pallas as pl
from jax.experimental.pallas import tpu as pltpu
from jax.experimental.pallas import tpu_sc as plsc

N = 10000
E = 320000
D = 128
NB = 5
NCMB = NB * NB * NB

NC = 2
NS = 16
NW = NC * NS
Q = E // NW
C = 80
IB = 25
NBLK = Q // (IB * C)
NPAIR = (IB - 1) // 2
NPR = C // 4
NPT = 624
NREM = N - NS * NPT


def _sc_agg_body(src_h, dst_h, pk_h, h_h, comb_h, out_h,
                 idx_s, idx_d, rows0, rows1, pk0, pk1, comb_v, agg_sh,
                 sg0, sg1, ss0, ss1, sp0, sp1):
    c = lax.axis_index("c")
    s = lax.axis_index("s")
    wid = c * NS + s

    pltpu.sync_copy(comb_h, comb_v)

    zero = jnp.zeros((16,), jnp.float32)

    def zrow(j, _):
        for k in range(D // 16):
            rows0[j, pl.ds(k * 16, 16)] = zero
        return 0

    lax.fori_loop(0, C, zrow, 0)

    def zcopy(j, _):
        pltpu.sync_copy(rows0, agg_sh.at[pl.ds(s * NPT + j * C, C)])
        return 0

    lax.fori_loop(0, NPT // C, zcopy, 0)
    pltpu.sync_copy(rows0.at[pl.ds(0, NPT - (NPT // C) * C)],
                    agg_sh.at[pl.ds(s * NPT + (NPT // C) * C,
                                    NPT - (NPT // C) * C)])

    @pl.when(s == 0)
    def _():
        pltpu.sync_copy(rows0.at[pl.ds(0, NREM)],
                        agg_sh.at[pl.ds(NS * NPT, NREM)])

    plsc.subcore_barrier()

    def compute_msgs(rows_v, pk_v, cc):
        @plsc.parallel_loop(0, C, 1, unroll=4)
        def _(j):
            row = cc * NPR + lax.div(j, 4)
            seg = lax.rem(j, 4) * 32
            wv = plsc.bitcast(pk_v[row, pl.ds(seg, 16)], jnp.float32)
            bv = pk_v[row, pl.ds(seg + 16, 16)]
            for k in range(D // 16):
                sl = pl.ds(k * 16, 16)
                r = rows_v[j, sl]
                cv = plsc.load_gather(comb_v, [bv + (k * 16)])
                rows_v[j, sl] = jnp.maximum(r + cv, 0.0) * wv

    def gather(i, rows_v, sem):
        pltpu.async_copy(h_h.at[idx_s.at[pl.ds(i * C, C)]], rows_v, sem)

    def wait_gather(i, rows_v, sem):
        pltpu.make_async_copy(h_h.at[idx_s.at[pl.ds(i * C, C)]], rows_v,
                              sem).wait()

    def scatter(i, rows_v, sem):
        pltpu.async_copy(rows_v, agg_sh.at[idx_d.at[i]], sem, add=True)

    def wait_scatter(i, rows_v, sem):
        pltpu.make_async_copy(rows_v, agg_sh.at[idx_d.at[i]], sem).wait()

    def pk_fetch(rp0, pk_v, sem):
        pltpu.async_copy(pk_h.at[pl.ds(rp0, 2 * NPR)], pk_v, sem)

    def wait_pk(rp0, pk_v, sem):
        pltpu.make_async_copy(pk_h.at[pl.ds(rp0, 2 * NPR)], pk_v,
                              sem).wait()

    def block_body(b, _):
        bi = wid * NBLK + b
        pltpu.sync_copy(src_h.at[pl.ds(bi * IB * C, IB * C)], idx_s)
        pltpu.sync_copy(dst_h.at[bi], idx_d)
        rb0 = bi * IB * NPR

        pk_fetch(rb0, pk0, sp0)
        gather(0, rows0, sg0)

        def run_pair(p, pk_v):
            i0 = 2 * p
            wait_gather(i0, rows0, sg0)

            @pl.when(p > 0)
            def _():
                wait_scatter(i0 - 1, rows1, ss1)

            gather(i0 + 1, rows1, sg1)
            compute_msgs(rows0, pk_v, 0)
            scatter(i0, rows0, ss0)
            wait_gather(i0 + 1, rows1, sg1)

            @pl.when(i0 + 2 < IB)
            def _():
                wait_scatter(i0, rows0, ss0)
                gather(i0 + 2, rows0, sg0)

            compute_msgs(rows1, pk_v, 1)
            scatter(i0 + 1, rows1, ss1)

        def pair2_body(q, _):
            p0 = 2 * q
            wait_pk(rb0 + p0 * 2 * NPR, pk0, sp0)
            pk_fetch(rb0 + (p0 + 1) * 2 * NPR, pk1, sp1)
            run_pair(p0, pk0)
            wait_pk(rb0 + (p0 + 1) * 2 * NPR, pk1, sp1)

            @pl.when(q + 1 < NPAIR // 2)
            def _():
                pk_fetch(rb0 + (p0 + 2) * 2 * NPR, pk0, sp0)

            run_pair(p0 + 1, pk1)
            return 0

        lax.fori_loop(0, NPAIR // 2, pair2_body, 0)
        pltpu.sync_copy(pk_h.at[pl.ds(rb0 + (IB - 1) * NPR, NPR)],
                        pk0.at[pl.ds(0, NPR)])
        wait_gather(IB - 1, rows0, sg0)
        compute_msgs(rows0, pk0, 0)
        scatter(IB - 1, rows0, ss0)
        wait_scatter(IB - 2, rows1, ss1)
        wait_scatter(IB - 1, rows0, ss0)
        return 0

    lax.fori_loop(0, NBLK, block_body, 0)
    plsc.subcore_barrier()
    pltpu.sync_copy(agg_sh.at[pl.ds(s * NPT, NPT)],
                    out_h.at[c, pl.ds(s * NPT, NPT)])

    @pl.when(s == 0)
    def _():
        pltpu.sync_copy(agg_sh.at[pl.ds(NS * NPT, NREM)],
                        out_h.at[c, pl.ds(NS * NPT, NREM)])


_sc_agg = pl.kernel(
    _sc_agg_body,
    out_type=jax.ShapeDtypeStruct((NC, N, D), jnp.float32),
    mesh=plsc.VectorSubcoreMesh(core_axis_name="c", subcore_axis_name="s",
                                num_cores=NC, num_subcores=NS),
    scratch_types=[
        pltpu.VMEM((IB * C,), jnp.int32),
        pltpu.VMEM((IB, C), jnp.int32),
        pltpu.VMEM((C, D), jnp.float32),
        pltpu.VMEM((C, D), jnp.float32),
        pltpu.VMEM((2 * NPR, D), jnp.int32),
        pltpu.VMEM((2 * NPR, D), jnp.int32),
        pltpu.VMEM((NCMB * D,), jnp.float32),
        pltpu.VMEM_SHARED((N, D), jnp.float32),
        pltpu.SemaphoreType.DMA,
        pltpu.SemaphoreType.DMA,
        pltpu.SemaphoreType.DMA,
        pltpu.SemaphoreType.DMA,
        pltpu.SemaphoreType.DMA,
        pltpu.SemaphoreType.DMA,
    ],
    compiler_params=pltpu.CompilerParams(use_tc_tiling_on_sc=False,
                                         needs_layout_passes=False),
)


def _comb_body(bt_ref, out_ref):
    nl = bt_ref.shape[0]
    for l in range(nl):
        b0 = bt_ref[l, 0]
        b1 = bt_ref[l, 1]
        b2 = bt_ref[l, 2]
        t = (b0[:, None, None, :] + b1[None, :, None, :]
             + b2[None, None, :, :]).reshape(NCMB, D)
        out_ref[l] = t


def _build_comb(bond_tab):
    nl = bond_tab.shape[0]
    return pl.pallas_call(
        _comb_body,
        out_shape=jax.ShapeDtypeStruct((nl, NCMB, D), jnp.float32),
    )(bond_tab)


def _mlp_body(last, h_ref, agg_ref, w1_ref, b1_ref, g1_ref, bb1_ref,
              w2_ref, b2_ref, go_ref, bo_ref, eps_ref, out_ref):
    h = h_ref[...]
    z = (1.0 + eps_ref[0]) * h + agg_ref[0] + agg_ref[1]
    u = jnp.dot(z, w1_ref[...], preferred_element_type=jnp.float32) + b1_ref[...]
    mu = jnp.mean(u, axis=0, keepdims=True)
    var = jnp.mean((u - mu) * (u - mu), axis=0, keepdims=True)
    u = (u - mu) * lax.rsqrt(var + 1e-5) * g1_ref[...] + bb1_ref[...]
    u = jnp.maximum(u, 0.0)
    v = jnp.dot(u, w2_ref[...], preferred_element_type=jnp.float32) + b2_ref[...]
    mu2 = jnp.mean(v, axis=0, keepdims=True)
    var2 = jnp.mean((v - mu2) * (v - mu2), axis=0, keepdims=True)
    v = (v - mu2) * lax.rsqrt(var2 + 1e-5) * go_ref[...] + bo_ref[...]
    if not last:
        v = jnp.maximum(v, 0.0)
    out_ref[...] = v


def _mlp(h, agg, w1, b1, g1, bb1, w2, b2, go, bo, eps_l, last):
    n, d = h.shape
    return pl.pallas_call(
        functools.partial(_mlp_body, last),
        out_shape=jax.ShapeDtypeStruct((n, d), jnp.float32),
        in_specs=[pl.BlockSpec(memory_space=pltpu.VMEM)] * 10
                 + [pl.BlockSpec(memory_space=pltpu.SMEM)],
    )(h, agg, w1, b1, g1, bb1, w2, b2, go, bo, eps_l)


def kernel(x, edge_index, edge_attr, edge_weights, bond_tab, W1, b1,
           bn1_g, bn1_b, W2, b2, eps, obn_g, obn_b):
    nl = W1.shape[0]
    src = edge_index[0]
    dst = edge_index[1]
    cmb = (edge_attr[:, 0] * NB + edge_attr[:, 1]) * NB + edge_attr[:, 2]

    dst3 = dst.reshape(NW * NBLK, IB, C)
    iota16 = jnp.arange(16, dtype=jnp.int32)
    wbits = jax.lax.bitcast_convert_type(edge_weights, jnp.int32)
    boff = cmb * D
    pieces = []
    for k in range(4):
        pieces.append(jnp.broadcast_to(wbits[k::4, None], (E // 4, 16)))
        pieces.append(boff[k::4, None] + iota16[None, :])
    pk = jnp.concatenate(pieces, axis=1)

    comb = _build_comb(bond_tab).reshape(nl, NCMB * D)

    h = x
    for l in range(nl):
        agg = _sc_agg(src, dst3, pk, h, comb[l])
        h = _mlp(h, agg, W1[l], b1[l][None, :], bn1_g[l][None, :],
                 bn1_b[l][None, :], W2[l], b2[l][None, :], obn_g[l][None, :],
                 obn_b[l][None, :], eps[l:l + 1], last=(l == nl - 1))
    return h


# speedup vs baseline: 3.5984x; 1.4877x over previous
import functools

import jax
import jax.numpy as jnp
from jax import lax
from jax.experimental import pallas as pl
from jax.experimental.pallas import tpu as pltpu
from jax.experimental.pallas import tpu_sc as plsc

N = 10000
E = 320000
D = 128
NB = 5
NCMB = NB * NB * NB

NC = 2
NS = 16
NW = NC * NS
Q = E // NW
C = 80
IB = 25
NBLK = Q // (IB * C)
NPAIR = (IB - 1) // 2
NPR = C // 4
NPT = 624
NREM = N - NS * NPT

import numpy as _np

_mw = _np.zeros((128, 4096), _np.float32)
_mb = _np.zeros((128, 4096), _np.float32)
for _c in range(4096):
    _e = _c // 32
    if _c % 32 < 16:
        _mw[_e, _c] = 1.0
    else:
        _mb[_e, _c] = 128.0
_MW = _mw
_MB = _mb
_IOTAROW = (_np.where(_np.arange(4096) % 32 >= 16, _np.arange(4096) % 16, 0)
            .astype(_np.float32)[None, :])


def _sc_agg_body(src_h, dst_h, pk_h, h_h, comb_h, out_h,
                 idx_s, idx_d, rows0, rows1, pk0, pk1, comb_v, agg_sh,
                 sg0, sg1, ss0, ss1, sp0, sp1):
    c = lax.axis_index("c")
    s = lax.axis_index("s")
    wid = c * NS + s

    pltpu.sync_copy(comb_h, comb_v)

    zero = jnp.zeros((16,), jnp.float32)

    def zrow(j, _):
        for k in range(D // 16):
            rows0[j, pl.ds(k * 16, 16)] = zero
        return 0

    lax.fori_loop(0, C, zrow, 0)

    def zcopy(j, _):
        pltpu.sync_copy(rows0, agg_sh.at[pl.ds(s * NPT + j * C, C)])
        return 0

    lax.fori_loop(0, NPT // C, zcopy, 0)
    pltpu.sync_copy(rows0.at[pl.ds(0, NPT - (NPT // C) * C)],
                    agg_sh.at[pl.ds(s * NPT + (NPT // C) * C,
                                    NPT - (NPT // C) * C)])

    @pl.when(s == 0)
    def _():
        pltpu.sync_copy(rows0.at[pl.ds(0, NREM)],
                        agg_sh.at[pl.ds(NS * NPT, NREM)])

    plsc.subcore_barrier()

    def compute_msgs(rows_v, pk_v, cc):
        @plsc.parallel_loop(0, C, 1, unroll=4)
        def _(j):
            base = (cc * C + j) * 32
            wv = pk_v[pl.ds(base, 16)]
            bv = pk_v[pl.ds(base + 16, 16)].astype(jnp.int32)
            for k in range(D // 16):
                sl = pl.ds(k * 16, 16)
                r = rows_v[j, sl]
                cv = plsc.load_gather(comb_v, [bv + (k * 16)])
                rows_v[j, sl] = jnp.maximum(r + cv, 0.0) * wv

    def gather(i, rows_v, sem):
        pltpu.async_copy(h_h.at[idx_s.at[pl.ds(i * C, C)]], rows_v, sem)

    def wait_gather(i, rows_v, sem):
        pltpu.make_async_copy(h_h.at[idx_s.at[pl.ds(i * C, C)]], rows_v,
                              sem).wait()

    def scatter(i, rows_v, sem):
        pltpu.async_copy(rows_v, agg_sh.at[idx_d.at[i]], sem, add=True)

    def wait_scatter(i, rows_v, sem):
        pltpu.make_async_copy(rows_v, agg_sh.at[idx_d.at[i]], sem).wait()

    def pk_fetch(cg, pk_v, sem):
        pltpu.async_copy(pk_h.at[pl.ds(cg * C * 32, 2 * C * 32)], pk_v, sem)

    def wait_pk(cg, pk_v, sem):
        pltpu.make_async_copy(pk_h.at[pl.ds(cg * C * 32, 2 * C * 32)], pk_v,
                              sem).wait()

    def block_body(b, _):
        bi = wid * NBLK + b
        pltpu.sync_copy(src_h.at[pl.ds(bi * IB * C, IB * C)], idx_s)
        pltpu.sync_copy(dst_h.at[bi], idx_d)
        cg0 = bi * IB

        pk_fetch(cg0, pk0, sp0)
        gather(0, rows0, sg0)

        def run_pair(p, pk_v):
            i0 = 2 * p
            wait_gather(i0, rows0, sg0)

            @pl.when(p > 0)
            def _():
                wait_scatter(i0 - 1, rows1, ss1)

            gather(i0 + 1, rows1, sg1)
            compute_msgs(rows0, pk_v, 0)
            scatter(i0, rows0, ss0)
            wait_gather(i0 + 1, rows1, sg1)

            @pl.when(i0 + 2 < IB)
            def _():
                wait_scatter(i0, rows0, ss0)
                gather(i0 + 2, rows0, sg0)

            compute_msgs(rows1, pk_v, 1)
            scatter(i0 + 1, rows1, ss1)

        def pair2_body(q, _):
            p0 = 2 * q
            wait_pk(cg0 + p0 * 2, pk0, sp0)
            pk_fetch(cg0 + (p0 + 1) * 2, pk1, sp1)
            run_pair(p0, pk0)
            wait_pk(cg0 + (p0 + 1) * 2, pk1, sp1)

            @pl.when(q + 1 < NPAIR // 2)
            def _():
                pk_fetch(cg0 + (p0 + 2) * 2, pk0, sp0)

            run_pair(p0 + 1, pk1)
            return 0

        lax.fori_loop(0, NPAIR // 2, pair2_body, 0)
        pltpu.sync_copy(pk_h.at[pl.ds((cg0 + IB - 1) * C * 32, C * 32)],
                        pk0.at[pl.ds(0, C * 32)])
        wait_gather(IB - 1, rows0, sg0)
        compute_msgs(rows0, pk0, 0)
        scatter(IB - 1, rows0, ss0)
        wait_scatter(IB - 2, rows1, ss1)
        wait_scatter(IB - 1, rows0, ss0)
        return 0

    lax.fori_loop(0, NBLK, block_body, 0)
    plsc.subcore_barrier()
    pltpu.sync_copy(agg_sh.at[pl.ds(s * NPT, NPT)],
                    out_h.at[c, pl.ds(s * NPT, NPT)])

    @pl.when(s == 0)
    def _():
        pltpu.sync_copy(agg_sh.at[pl.ds(NS * NPT, NREM)],
                        out_h.at[c, pl.ds(NS * NPT, NREM)])


_sc_agg = pl.kernel(
    _sc_agg_body,
    out_type=jax.ShapeDtypeStruct((NC, N, D), jnp.float32),
    mesh=plsc.VectorSubcoreMesh(core_axis_name="c", subcore_axis_name="s",
                                num_cores=NC, num_subcores=NS),
    scratch_types=[
        pltpu.VMEM((IB * C,), jnp.int32),
        pltpu.VMEM((IB, C), jnp.int32),
        pltpu.VMEM((C, D), jnp.float32),
        pltpu.VMEM((C, D), jnp.float32),
        pltpu.VMEM((2 * C * 32,), jnp.float32),
        pltpu.VMEM((2 * C * 32,), jnp.float32),
        pltpu.VMEM((NCMB * D,), jnp.float32),
        pltpu.VMEM_SHARED((N, D), jnp.float32),
        pltpu.SemaphoreType.DMA,
        pltpu.SemaphoreType.DMA,
        pltpu.SemaphoreType.DMA,
        pltpu.SemaphoreType.DMA,
        pltpu.SemaphoreType.DMA,
        pltpu.SemaphoreType.DMA,
    ],
    compiler_params=pltpu.CompilerParams(use_tc_tiling_on_sc=False,
                                         needs_layout_passes=False),
)


def _comb_body(bt_ref, out_ref):
    nl = bt_ref.shape[0]
    for l in range(nl):
        b0 = bt_ref[l, 0]
        b1 = bt_ref[l, 1]
        b2 = bt_ref[l, 2]
        t = (b0[:, None, None, :] + b1[None, :, None, :]
             + b2[None, None, :, :]).reshape(NCMB, D)
        out_ref[l] = t


def _build_comb(bond_tab):
    nl = bond_tab.shape[0]
    return pl.pallas_call(
        _comb_body,
        out_shape=jax.ShapeDtypeStruct((nl, NCMB, D), jnp.float32),
    )(bond_tab)


def _mlp_body(last, h_ref, agg_ref, w1_ref, b1_ref, g1_ref, bb1_ref,
              w2_ref, b2_ref, go_ref, bo_ref, eps_ref, out_ref):
    h = h_ref[...]
    z = (1.0 + eps_ref[0]) * h + agg_ref[0] + agg_ref[1]
    u = jnp.dot(z, w1_ref[...], preferred_element_type=jnp.float32) + b1_ref[...]
    mu = jnp.mean(u, axis=0, keepdims=True)
    var = jnp.mean((u - mu) * (u - mu), axis=0, keepdims=True)
    u = (u - mu) * lax.rsqrt(var + 1e-5) * g1_ref[...] + bb1_ref[...]
    u = jnp.maximum(u, 0.0)
    v = jnp.dot(u, w2_ref[...], preferred_element_type=jnp.float32) + b2_ref[...]
    mu2 = jnp.mean(v, axis=0, keepdims=True)
    var2 = jnp.mean((v - mu2) * (v - mu2), axis=0, keepdims=True)
    v = (v - mu2) * lax.rsqrt(var2 + 1e-5) * go_ref[...] + bo_ref[...]
    if not last:
        v = jnp.maximum(v, 0.0)
    out_ref[...] = v


def _mlp(h, agg, w1, b1, g1, bb1, w2, b2, go, bo, eps_l, last):
    n, d = h.shape
    return pl.pallas_call(
        functools.partial(_mlp_body, last),
        out_shape=jax.ShapeDtypeStruct((n, d), jnp.float32),
        in_specs=[pl.BlockSpec(memory_space=pltpu.VMEM)] * 10
                 + [pl.BlockSpec(memory_space=pltpu.SMEM)],
    )(h, agg, w1, b1, g1, bb1, w2, b2, go, bo, eps_l)


def kernel(x, edge_index, edge_attr, edge_weights, bond_tab, W1, b1,
           bn1_g, bn1_b, W2, b2, eps, obn_g, obn_b):
    nl = W1.shape[0]
    src = edge_index[0]
    dst = edge_index[1]
    cmb = (edge_attr[:, 0] * NB + edge_attr[:, 1]) * NB + edge_attr[:, 2]

    dst3 = dst.reshape(NW * NBLK, IB, C)
    w2 = edge_weights.reshape(E // 128, 128)
    cmb2 = cmb.astype(jnp.float32).reshape(E // 128, 128)
    pk = (jnp.dot(w2, _MW, precision=jax.lax.Precision.HIGHEST)
          + jnp.dot(cmb2, _MB, precision=jax.lax.Precision.HIGHEST)
          + _IOTAROW).reshape(E * 32)

    comb = _build_comb(bond_tab).reshape(nl, NCMB * D)

    h = x
    for l in range(nl):
        agg = _sc_agg(src, dst3, pk, h, comb[l])
        h = _mlp(h, agg, W1[l], b1[l][None, :], bn1_g[l][None, :],
                 bn1_b[l][None, :], W2[l], b2[l][None, :], obn_g[l][None, :],
                 obn_b[l][None, :], eps[l:l + 1], last=(l == nl - 1))
    return h
